# Initial kernel scaffold; baseline (speedup 1.0000x reference)
import jax
import jax.numpy as jnp
from jax.experimental import pallas as pl


def kernel(feature, table, Wp, bp, Wm, bm, Wo, bo):
    raise NotImplementedError("write your pallas kernel here")


# speedup vs baseline: 3.6698x; 3.6698x over previous
import functools

import jax
import jax.numpy as jnp
from jax.experimental import pallas as pl
from jax.experimental.pallas import tpu as pltpu
from jax.experimental.pallas import tpu_sc as plsc

_COLUMNS = 1000000
_EMBED = 32
_POS = 40
_HID = 256
_BATCH = 16384
_D = _EMBED * _POS

_N_IDX = _BATCH * _POS
_WINDOW = 128

_TILE = 256


def _sc_gather(table128, idx_flat):
    mesh = plsc.VectorSubcoreMesh(core_axis_name="c", subcore_axis_name="s")

    @functools.partial(
        pl.kernel,
        out_type=jax.ShapeDtypeStruct((_N_IDX, 128), jnp.float32),
        mesh=mesh,
    )
    def k(table_hbm, i_hbm, o_hbm):
        def body(i_vmem, o_vmem):
            pltpu.sync_copy(table_hbm.at[i_vmem.at[0]], o_vmem)

        pltpu.emit_pipeline(
            body,
            grid=(_N_IDX // _WINDOW,),
            in_specs=[pl.BlockSpec((1, _WINDOW), index_map=lambda i: (0, i))],
            out_specs=[pl.BlockSpec((_WINDOW, 128), index_map=lambda i: (i, 0))],
            core_axis_name=("c", "s"),
            dimension_semantics=(pltpu.PARALLEL,),
        )(i_hbm, o_hbm)

    return k(table128, idx_flat.reshape(1, _N_IDX))


def _leaky(x):
    return jnp.where(x >= 0, x, 0.01 * x)


def _mlp_body(g_ref, o2_ref, wp_ref, bp_ref, wm_ref, bm_ref, wo_ref, bo_ref,
              out_ref, x_ref):
    for p in range(_POS):
        off = o2_ref[:, p:p + 1]
        w = g_ref[:, p, 0:_EMBED] * (off == 0).astype(jnp.float32)
        for kk in range(1, 4):
            w = w + (g_ref[:, p, kk * _EMBED:(kk + 1) * _EMBED]
                     * (off == kk).astype(jnp.float32))
        x_ref[:, p * _EMBED:(p + 1) * _EMBED] = w
    x = x_ref[...]
    h = _leaky(jnp.dot(x, wp_ref[...], preferred_element_type=jnp.float32)
               + bp_ref[...])
    h = _leaky(jnp.dot(h, wm_ref[...], preferred_element_type=jnp.float32)
               + bm_ref[...])
    out_ref[...] = _leaky(
        jnp.dot(h, wo_ref[...], preferred_element_type=jnp.float32)
        + bo_ref[...])


def _mlp(g3, o2, wp_t, bp, wm_t, bm, wo_t, bo):
    grid = (_BATCH // _TILE,)
    return pl.pallas_call(
        _mlp_body,
        grid=grid,
        in_specs=[
            pl.BlockSpec((_TILE, _POS, 128), lambda i: (i, 0, 0)),
            pl.BlockSpec((_TILE, _POS), lambda i: (i, 0)),
            pl.BlockSpec((_D, _D), lambda i: (0, 0)),
            pl.BlockSpec((1, _D), lambda i: (0, 0)),
            pl.BlockSpec((_D, _HID), lambda i: (0, 0)),
            pl.BlockSpec((1, _HID), lambda i: (0, 0)),
            pl.BlockSpec((_HID, _EMBED), lambda i: (0, 0)),
            pl.BlockSpec((1, _EMBED), lambda i: (0, 0)),
        ],
        out_specs=pl.BlockSpec((_TILE, _EMBED), lambda i: (i, 0)),
        out_shape=jax.ShapeDtypeStruct((_BATCH, _EMBED), jnp.float32),
        scratch_shapes=[pltpu.VMEM((_TILE, _D), jnp.float32)],
    )(g3, o2, wp_t, bp, wm_t, bm, wo_t, bo)


def kernel(feature, table, Wp, bp, Wm, bm, Wo, bo):
    feat = feature.astype(jnp.int32)
    idx4 = (feat // 4).reshape(-1)
    o2 = feat % 4
    table128 = table.reshape(_COLUMNS // 4, 128)
    g = _sc_gather(table128, idx4)
    g3 = g.reshape(_BATCH, _POS, 128)
    out = _mlp(g3, o2, Wp.T, bp.reshape(1, _D), Wm.T, bm.reshape(1, _HID),
               Wo.T, bo.reshape(1, _EMBED))
    return out


# speedup vs baseline: 11.4716x; 3.1259x over previous
import functools

import jax
import jax.numpy as jnp
from jax.experimental import pallas as pl
from jax.experimental.pallas import tpu as pltpu
from jax.experimental.pallas import tpu_sc as plsc

_COLUMNS = 1000000
_EMBED = 32
_POS = 40
_HID = 256
_BATCH = 16384
_D = _EMBED * _POS

_N_IDX = _BATCH * _POS
_WINDOW = 128

_TILE = 256


def _sc_gather(table128, idx_flat):
    mesh = plsc.VectorSubcoreMesh(core_axis_name="c", subcore_axis_name="s")

    @functools.partial(
        pl.kernel,
        out_type=jax.ShapeDtypeStruct((_N_IDX, 128), jnp.float32),
        mesh=mesh,
    )
    def k(table_hbm, i_hbm, o_hbm):
        def body(i_vmem, o_vmem):
            pltpu.sync_copy(table_hbm.at[i_vmem.at[0]], o_vmem)

        pltpu.emit_pipeline(
            body,
            grid=(_N_IDX // _WINDOW,),
            in_specs=[pl.BlockSpec((1, _WINDOW), index_map=lambda i: (0, i))],
            out_specs=[pl.BlockSpec((_WINDOW, 128), index_map=lambda i: (i, 0))],
            core_axis_name=("c", "s"),
            dimension_semantics=(pltpu.PARALLEL,),
        )(i_hbm, o_hbm)

    return k(table128, idx_flat.reshape(1, _N_IDX))


def _leaky(x):
    return jnp.where(x >= 0, x, 0.01 * x)


def _mlp_body(g_ref, o2_ref, wp_ref, bp_ref, wm_ref, bm_ref, wo_ref, bo_ref,
              out_ref, x_ref):
    wid = jax.lax.broadcasted_iota(jnp.int32, (1, 128), 1) // _EMBED
    for p in range(_POS):
        off = o2_ref[:, p:p + 1]
        t = jnp.where(off == wid, g_ref[:, p, :], 0.0)
        w = (t[:, 0:32] + t[:, 32:64]) + (t[:, 64:96] + t[:, 96:128])
        x_ref[:, p * _EMBED:(p + 1) * _EMBED] = w.astype(jnp.bfloat16)
    x = x_ref[...]
    h = _leaky(jnp.dot(x, wp_ref[...], preferred_element_type=jnp.float32)
               + bp_ref[...])
    h = _leaky(jnp.dot(h.astype(jnp.bfloat16), wm_ref[...],
                       preferred_element_type=jnp.float32) + bm_ref[...])
    out_ref[...] = _leaky(
        jnp.dot(h.astype(jnp.bfloat16), wo_ref[...],
                preferred_element_type=jnp.float32) + bo_ref[...])


def _mlp(g3, o2, wp_t, bp, wm_t, bm, wo_t, bo):
    grid = (_BATCH // _TILE,)
    return pl.pallas_call(
        _mlp_body,
        grid=grid,
        in_specs=[
            pl.BlockSpec((_TILE, _POS, 128), lambda i: (i, 0, 0)),
            pl.BlockSpec((_TILE, _POS), lambda i: (i, 0)),
            pl.BlockSpec((_D, _D), lambda i: (0, 0)),
            pl.BlockSpec((1, _D), lambda i: (0, 0)),
            pl.BlockSpec((_D, _HID), lambda i: (0, 0)),
            pl.BlockSpec((1, _HID), lambda i: (0, 0)),
            pl.BlockSpec((_HID, _EMBED), lambda i: (0, 0)),
            pl.BlockSpec((1, _EMBED), lambda i: (0, 0)),
        ],
        out_specs=pl.BlockSpec((_TILE, _EMBED), lambda i: (i, 0)),
        out_shape=jax.ShapeDtypeStruct((_BATCH, _EMBED), jnp.float32),
        scratch_shapes=[pltpu.VMEM((_TILE, _D), jnp.bfloat16)],
    )(g3, o2, wp_t, bp, wm_t, bm, wo_t, bo)


def kernel(feature, table, Wp, bp, Wm, bm, Wo, bo):
    feat = feature.astype(jnp.int32)
    idx4 = (feat // 4).reshape(-1)
    o2 = feat % 4
    table128 = table.reshape(_COLUMNS // 4, 128)
    g = _sc_gather(table128, idx4)
    g3 = g.reshape(_BATCH, _POS, 128)
    out = _mlp(g3, o2,
               Wp.T.astype(jnp.bfloat16), bp.reshape(1, _D),
               Wm.T.astype(jnp.bfloat16), bm.reshape(1, _HID),
               Wo.T.astype(jnp.bfloat16), bo.reshape(1, _EMBED))
    return out


# speedup vs baseline: 13.2821x; 1.1578x over previous
import functools

import jax
import jax.numpy as jnp
from jax.experimental import pallas as pl
from jax.experimental.pallas import tpu as pltpu
from jax.experimental.pallas import tpu_sc as plsc

_COLUMNS = 1000000
_EMBED = 32
_POS = 40
_HID = 256
_BATCH = 16384
_D = _EMBED * _POS

_N_IDX = _BATCH * _POS
_WINDOW = 128

_TILE = 256
_NCHUNK = 4
_CB = _BATCH // _NCHUNK


def _sc_gather(table128, idx_flat, n_idx):
    mesh = plsc.VectorSubcoreMesh(core_axis_name="c", subcore_axis_name="s")

    @functools.partial(
        pl.kernel,
        out_type=jax.ShapeDtypeStruct((n_idx, 128), jnp.float32),
        mesh=mesh,
    )
    def k(table_hbm, i_hbm, o_hbm):
        def body(i_vmem, o_vmem):
            pltpu.sync_copy(table_hbm.at[i_vmem.at[0]], o_vmem)

        pltpu.emit_pipeline(
            body,
            grid=(n_idx // _WINDOW,),
            in_specs=[pl.BlockSpec((1, _WINDOW), index_map=lambda i: (0, i))],
            out_specs=[pl.BlockSpec((_WINDOW, 128), index_map=lambda i: (i, 0))],
            core_axis_name=("c", "s"),
            dimension_semantics=(pltpu.PARALLEL,),
        )(i_hbm, o_hbm)

    return k(table128, idx_flat.reshape(1, n_idx))


def _leaky(x):
    return jnp.where(x >= 0, x, 0.01 * x)


def _mlp_body(g_ref, o2_ref, wp_ref, bp_ref, wm_ref, bm_ref, wo_ref, bo_ref,
              out_ref, x_ref):
    wid = jax.lax.broadcasted_iota(jnp.int32, (1, 128), 1) // _EMBED
    for p in range(_POS):
        off = o2_ref[:, p:p + 1]
        t = jnp.where(off == wid, g_ref[:, p, :], 0.0)
        w = (t[:, 0:32] + t[:, 32:64]) + (t[:, 64:96] + t[:, 96:128])
        x_ref[:, p * _EMBED:(p + 1) * _EMBED] = w.astype(jnp.bfloat16)
    x = x_ref[...]
    h = _leaky(jnp.dot(x, wp_ref[...], preferred_element_type=jnp.float32)
               + bp_ref[...])
    h = _leaky(jnp.dot(h.astype(jnp.bfloat16), wm_ref[...],
                       preferred_element_type=jnp.float32) + bm_ref[...])
    out_ref[...] = _leaky(
        jnp.dot(h.astype(jnp.bfloat16), wo_ref[...],
                preferred_element_type=jnp.float32) + bo_ref[...])


def _mlp(g3, o2, wp_t, bp, wm_t, bm, wo_t, bo, batch):
    grid = (batch // _TILE,)
    return pl.pallas_call(
        _mlp_body,
        grid=grid,
        in_specs=[
            pl.BlockSpec((_TILE, _POS, 128), lambda i: (i, 0, 0)),
            pl.BlockSpec((_TILE, _POS), lambda i: (i, 0)),
            pl.BlockSpec((_D, _D), lambda i: (0, 0)),
            pl.BlockSpec((1, _D), lambda i: (0, 0)),
            pl.BlockSpec((_D, _HID), lambda i: (0, 0)),
            pl.BlockSpec((1, _HID), lambda i: (0, 0)),
            pl.BlockSpec((_HID, _EMBED), lambda i: (0, 0)),
            pl.BlockSpec((1, _EMBED), lambda i: (0, 0)),
        ],
        out_specs=pl.BlockSpec((_TILE, _EMBED), lambda i: (i, 0)),
        out_shape=jax.ShapeDtypeStruct((batch, _EMBED), jnp.float32),
        scratch_shapes=[pltpu.VMEM((_TILE, _D), jnp.bfloat16)],
    )(g3, o2, wp_t, bp, wm_t, bm, wo_t, bo)


def kernel(feature, table, Wp, bp, Wm, bm, Wo, bo):
    feat = feature.astype(jnp.int32)
    table128 = table.reshape(_COLUMNS // 4, 128)
    wp = Wp.T.astype(jnp.bfloat16)
    wm = Wm.T.astype(jnp.bfloat16)
    wo = Wo.T.astype(jnp.bfloat16)
    bp2 = bp.reshape(1, _D)
    bm2 = bm.reshape(1, _HID)
    bo2 = bo.reshape(1, _EMBED)
    outs = []
    for c in range(_NCHUNK):
        fc = feat[c * _CB:(c + 1) * _CB]
        idx4 = (fc // 4).reshape(-1)
        o2 = fc % 4
        g = _sc_gather(table128, idx4, _CB * _POS)
        g3 = g.reshape(_CB, _POS, 128)
        outs.append(_mlp(g3, o2, wp, bp2, wm, bm2, wo, bo2, _CB))
    return jnp.concatenate(outs, axis=0)


# speedup vs baseline: 13.3166x; 1.0026x over previous
import functools

import jax
import jax.numpy as jnp
from jax.experimental import pallas as pl
from jax.experimental.pallas import tpu as pltpu
from jax.experimental.pallas import tpu_sc as plsc

_COLUMNS = 1000000
_EMBED = 32
_POS = 40
_HID = 256
_BATCH = 16384
_D = _EMBED * _POS

_N_IDX = _BATCH * _POS
_WINDOW = 128

_TILE = 256
_NCHUNK = 4
_CB = _BATCH // _NCHUNK


def _sc_gather(table128, idx_flat, n_idx):
    mesh = plsc.VectorSubcoreMesh(core_axis_name="c", subcore_axis_name="s")

    @functools.partial(
        pl.kernel,
        out_type=jax.ShapeDtypeStruct((n_idx, 128), jnp.float32),
        mesh=mesh,
    )
    def k(table_hbm, i_hbm, o_hbm):
        def body(i_vmem, o_vmem):
            pltpu.sync_copy(table_hbm.at[i_vmem.at[0]], o_vmem)

        pltpu.emit_pipeline(
            body,
            grid=(n_idx // _WINDOW,),
            in_specs=[pl.BlockSpec((1, _WINDOW), index_map=lambda i: (0, i))],
            out_specs=[pl.BlockSpec((_WINDOW, 128), index_map=lambda i: (i, 0))],
            core_axis_name=("c", "s"),
            dimension_semantics=(pltpu.PARALLEL,),
        )(i_hbm, o_hbm)

    return k(table128, idx_flat.reshape(1, n_idx))


def _leaky(x):
    return jnp.where(x >= 0, x, 0.01 * x)


def _mlp_body(g_ref, o2_ref, wp_ref, bp_ref, wm_ref, bm_ref, wo_ref, bo_ref,
              out_ref, x_ref):
    wid = jax.lax.broadcasted_iota(jnp.int32, (1, 128), 1) // _EMBED
    for p in range(_POS):
        off = o2_ref[:, p:p + 1]
        t = jnp.where(off == wid, g_ref[:, p, :], 0.0)
        w = (t[:, 0:32] + t[:, 32:64]) + (t[:, 64:96] + t[:, 96:128])
        x_ref[:, p * _EMBED:(p + 1) * _EMBED] = w.astype(jnp.bfloat16)
    x = x_ref[...]
    dnums = (((1,), (1,)), ((), ()))
    h = _leaky(jax.lax.dot_general(x, wp_ref[...], dnums,
                                   preferred_element_type=jnp.float32)
               + bp_ref[...])
    h = _leaky(jax.lax.dot_general(h.astype(jnp.bfloat16), wm_ref[...], dnums,
                                   preferred_element_type=jnp.float32)
               + bm_ref[...])
    out_ref[...] = _leaky(
        jax.lax.dot_general(h.astype(jnp.bfloat16), wo_ref[...], dnums,
                            preferred_element_type=jnp.float32) + bo_ref[...])


def _mlp(g3, o2, wp_t, bp, wm_t, bm, wo_t, bo, batch):
    grid = (batch // _TILE,)
    return pl.pallas_call(
        _mlp_body,
        grid=grid,
        in_specs=[
            pl.BlockSpec((_TILE, _POS, 128), lambda i: (i, 0, 0)),
            pl.BlockSpec((_TILE, _POS), lambda i: (i, 0)),
            pl.BlockSpec((_D, _D), lambda i: (0, 0)),
            pl.BlockSpec((1, _D), lambda i: (0, 0)),
            pl.BlockSpec((_HID, _D), lambda i: (0, 0)),
            pl.BlockSpec((1, _HID), lambda i: (0, 0)),
            pl.BlockSpec((_EMBED, _HID), lambda i: (0, 0)),
            pl.BlockSpec((1, _EMBED), lambda i: (0, 0)),
        ],
        out_specs=pl.BlockSpec((_TILE, _EMBED), lambda i: (i, 0)),
        out_shape=jax.ShapeDtypeStruct((batch, _EMBED), jnp.float32),
        scratch_shapes=[pltpu.VMEM((_TILE, _D), jnp.bfloat16)],
    )(g3, o2, wp_t, bp, wm_t, bm, wo_t, bo)


def kernel(feature, table, Wp, bp, Wm, bm, Wo, bo):
    feat = feature.astype(jnp.int32)
    table128 = table.reshape(_COLUMNS // 4, 128)
    wp = Wp.astype(jnp.bfloat16)
    wm = Wm.astype(jnp.bfloat16)
    wo = Wo.astype(jnp.bfloat16)
    bp2 = bp.reshape(1, _D)
    bm2 = bm.reshape(1, _HID)
    bo2 = bo.reshape(1, _EMBED)
    outs = []
    for c in range(_NCHUNK):
        fc = feat[c * _CB:(c + 1) * _CB]
        idx4 = (fc // 4).reshape(-1)
        o2 = fc % 4
        g = _sc_gather(table128, idx4, _CB * _POS)
        g3 = g.reshape(_CB, _POS, 128)
        outs.append(_mlp(g3, o2, wp, bp2, wm, bm2, wo, bo2, _CB))
    return jnp.concatenate(outs, axis=0)


# speedup vs baseline: 15.8942x; 1.1936x over previous
import functools

import jax
import jax.numpy as jnp
from jax.experimental import pallas as pl
from jax.experimental.pallas import tpu as pltpu
from jax.experimental.pallas import tpu_sc as plsc

_COLUMNS = 1000000
_EMBED = 32
_POS = 40
_HID = 256
_BATCH = 16384
_D = _EMBED * _POS

_N_IDX = _BATCH * _POS
_WINDOW = 128

_TILE = 256
_NCHUNK = 4
_CB = _BATCH // _NCHUNK


def _sc_gather(table128, idx_flat, n_idx):
    mesh = plsc.VectorSubcoreMesh(core_axis_name="c", subcore_axis_name="s")

    @functools.partial(
        pl.kernel,
        out_type=jax.ShapeDtypeStruct((n_idx, 128), jnp.float32),
        mesh=mesh,
    )
    def k(table_hbm, i_hbm, o_hbm):
        def body(i_vmem, o_vmem):
            pltpu.sync_copy(table_hbm.at[i_vmem.at[0]], o_vmem)

        pltpu.emit_pipeline(
            body,
            grid=(n_idx // _WINDOW,),
            in_specs=[pl.BlockSpec((1, _WINDOW), index_map=lambda i: (0, i))],
            out_specs=[pl.BlockSpec((_WINDOW, 128), index_map=lambda i: (i, 0))],
            core_axis_name=("c", "s"),
            dimension_semantics=(pltpu.PARALLEL,),
        )(i_hbm, o_hbm)

    return k(table128, idx_flat.reshape(1, n_idx))


def _leaky(x):
    return jnp.where(x >= 0, x, 0.01 * x)


def _mlp_body(g_ref, o2_ref, s_ref, wp_ref, bp_ref, wm_ref, bm_ref, wo_ref,
              bo_ref, out_ref, x_ref):
    wid = jax.lax.broadcasted_iota(jnp.int32, (1, 128), 1) // _EMBED
    for p in range(_POS):
        off = o2_ref[:, p:p + 1]
        t = jnp.where(off == wid, g_ref[:, p, :], 0.0).astype(jnp.bfloat16)
        w = jnp.dot(t, s_ref[...], preferred_element_type=jnp.float32)
        x_ref[:, p * _EMBED:(p + 1) * _EMBED] = w.astype(jnp.bfloat16)
    x = x_ref[...]
    dnums = (((1,), (1,)), ((), ()))
    h = _leaky(jax.lax.dot_general(x, wp_ref[...], dnums,
                                   preferred_element_type=jnp.float32)
               + bp_ref[...])
    h = _leaky(jax.lax.dot_general(h.astype(jnp.bfloat16), wm_ref[...], dnums,
                                   preferred_element_type=jnp.float32)
               + bm_ref[...])
    out_ref[...] = _leaky(
        jax.lax.dot_general(h.astype(jnp.bfloat16), wo_ref[...], dnums,
                            preferred_element_type=jnp.float32) + bo_ref[...])


def _mlp(g3, o2, sel, wp_t, bp, wm_t, bm, wo_t, bo, batch):
    grid = (batch // _TILE,)
    return pl.pallas_call(
        _mlp_body,
        grid=grid,
        in_specs=[
            pl.BlockSpec((_TILE, _POS, 128), lambda i: (i, 0, 0)),
            pl.BlockSpec((_TILE, _POS), lambda i: (i, 0)),
            pl.BlockSpec((128, _EMBED), lambda i: (0, 0)),
            pl.BlockSpec((_D, _D), lambda i: (0, 0)),
            pl.BlockSpec((1, _D), lambda i: (0, 0)),
            pl.BlockSpec((_HID, _D), lambda i: (0, 0)),
            pl.BlockSpec((1, _HID), lambda i: (0, 0)),
            pl.BlockSpec((_EMBED, _HID), lambda i: (0, 0)),
            pl.BlockSpec((1, _EMBED), lambda i: (0, 0)),
        ],
        out_specs=pl.BlockSpec((_TILE, _EMBED), lambda i: (i, 0)),
        out_shape=jax.ShapeDtypeStruct((batch, _EMBED), jnp.float32),
        scratch_shapes=[pltpu.VMEM((_TILE, _D), jnp.bfloat16)],
    )(g3, o2, sel, wp_t, bp, wm_t, bm, wo_t, bo)


def kernel(feature, table, Wp, bp, Wm, bm, Wo, bo):
    feat = feature.astype(jnp.int32)
    table128 = table.reshape(_COLUMNS // 4, 128)
    wp = Wp.astype(jnp.bfloat16)
    wm = Wm.astype(jnp.bfloat16)
    wo = Wo.astype(jnp.bfloat16)
    bp2 = bp.reshape(1, _D)
    bm2 = bm.reshape(1, _HID)
    bo2 = bo.reshape(1, _EMBED)
    sel = jnp.tile(jnp.eye(_EMBED, dtype=jnp.bfloat16), (4, 1))
    outs = []
    for c in range(_NCHUNK):
        fc = feat[c * _CB:(c + 1) * _CB]
        idx4 = (fc // 4).reshape(-1)
        o2 = fc % 4
        g = _sc_gather(table128, idx4, _CB * _POS)
        g3 = g.reshape(_CB, _POS, 128)
        outs.append(_mlp(g3, o2, sel, wp, bp2, wm, bm2, wo, bo2, _CB))
    return jnp.concatenate(outs, axis=0)
